# baseline (speedup 1.0000x reference)
import functools

import jax
import jax.numpy as jnp
from jax import lax
from jax.experimental import pallas as pl
from jax.experimental.pallas import tpu as pltpu
from jax.experimental.pallas import tpu_sc as plsc

CHUNK = 128
NUM_WORKERS = 32


def _make_lookup(chunks_per_worker, embed):
    rows_per_worker = chunks_per_worker * CHUNK
    total_rows = NUM_WORKERS * rows_per_worker

    mesh = plsc.VectorSubcoreMesh(core_axis_name="c", subcore_axis_name="s")

    assert chunks_per_worker % 2 == 0 and chunks_per_worker >= 4

    @functools.partial(
        pl.kernel,
        out_type=jax.ShapeDtypeStruct((total_rows, embed), jnp.float32),
        mesh=mesh,
        scratch_types=[
            pltpu.VMEM((chunks_per_worker, CHUNK), jnp.int32),
            pltpu.VMEM((CHUNK, embed), jnp.float32),
            pltpu.VMEM((CHUNK, embed), jnp.float32),
            pltpu.SemaphoreType.DMA,
            pltpu.SemaphoreType.DMA,
        ],
    )
    def lookup(x_hbm, table_hbm, out_hbm, idx_v, rows_a, rows_b, sem_a, sem_b):
        num_cores = 2
        wid = lax.axis_index("s") * num_cores + lax.axis_index("c")
        base = wid * rows_per_worker

        pltpu.sync_copy(x_hbm.at[pl.ds(wid * chunks_per_worker, chunks_per_worker)], idx_v)

        def gather_start(j, buf, sem):
            pltpu.async_copy(table_hbm.at[idx_v.at[j]], buf, sem)

        def gather_wait(j, buf, sem):
            pltpu.make_async_copy(table_hbm.at[idx_v.at[j]], buf, sem).wait()

        def writeback(j, buf):
            pltpu.sync_copy(buf, out_hbm.at[pl.ds(base + j * CHUNK, CHUNK)])

        gather_start(0, rows_a, sem_a)
        gather_start(1, rows_b, sem_b)

        def body(jj, carry):
            j0 = 2 * jj
            gather_wait(j0, rows_a, sem_a)
            writeback(j0, rows_a)
            gather_start(j0 + 2, rows_a, sem_a)
            gather_wait(j0 + 1, rows_b, sem_b)
            writeback(j0 + 1, rows_b)
            gather_start(j0 + 3, rows_b, sem_b)
            return carry

        last = chunks_per_worker - 2
        lax.fori_loop(0, chunks_per_worker // 2 - 1, body, 0)
        gather_wait(last, rows_a, sem_a)
        writeback(last, rows_a)
        gather_wait(last + 1, rows_b, sem_b)
        writeback(last + 1, rows_b)

    return lookup


def kernel(x, table):
    batch, hist = x.shape
    vocab, embed = table.shape
    hist_pad = -(-hist // 8) * 8
    total_pad = batch * hist_pad
    assert total_pad % (NUM_WORKERS * CHUNK) == 0
    chunks_per_worker = total_pad // (NUM_WORKERS * CHUNK)

    xp = jnp.pad(x.astype(jnp.int32), ((0, 0), (0, hist_pad - hist)))
    xp = xp.reshape(total_pad // CHUNK, CHUNK)
    lookup = _make_lookup(chunks_per_worker, embed)
    out = lookup(xp, table)
    return out.reshape(batch, hist_pad, embed)[:, :hist, :]


# speedup vs baseline: 6.6008x; 6.6008x over previous
import functools

import jax
import jax.numpy as jnp
from jax import lax
from jax.experimental import pallas as pl
from jax.experimental.pallas import tpu as pltpu
from jax.experimental.pallas import tpu_sc as plsc

CHUNK = 128
NUM_WORKERS = 32


def _make_lookup(chunks_per_worker, embed):
    rows_per_worker = chunks_per_worker * CHUNK
    total_rows = NUM_WORKERS * rows_per_worker

    mesh = plsc.VectorSubcoreMesh(core_axis_name="c", subcore_axis_name="s")

    assert chunks_per_worker % 2 == 0 and chunks_per_worker >= 4

    @functools.partial(
        pl.kernel,
        out_type=jax.ShapeDtypeStruct((total_rows, embed), jnp.float32),
        mesh=mesh,
        scratch_types=[
            pltpu.VMEM((chunks_per_worker, CHUNK), jnp.int32),
            pltpu.VMEM((CHUNK, embed), jnp.float32),
            pltpu.VMEM((CHUNK, embed), jnp.float32),
            pltpu.SemaphoreType.DMA,
            pltpu.SemaphoreType.DMA,
        ],
    )
    def lookup(x_hbm, table_hbm, out_hbm, idx_v, rows_a, rows_b, sem_a, sem_b):
        num_cores = 2
        wid = lax.axis_index("s") * num_cores + lax.axis_index("c")
        base = wid * rows_per_worker

        pltpu.sync_copy(x_hbm.at[pl.ds(wid * chunks_per_worker, chunks_per_worker)], idx_v)

        def gather_start(j, buf, sem):
            pltpu.async_copy(table_hbm.at[idx_v.at[j]], buf, sem)

        def gather_wait(j, buf, sem):
            pltpu.make_async_copy(table_hbm.at[idx_v.at[j]], buf, sem).wait()

        def writeback(j, buf):
            pltpu.sync_copy(buf, out_hbm.at[pl.ds(base + j * CHUNK, CHUNK)])

        gather_start(0, rows_a, sem_a)
        gather_start(1, rows_b, sem_b)

        def body(jj, carry):
            j0 = 2 * jj
            gather_wait(j0, rows_a, sem_a)
            writeback(j0, rows_a)
            gather_start(j0 + 2, rows_a, sem_a)
            gather_wait(j0 + 1, rows_b, sem_b)
            writeback(j0 + 1, rows_b)
            gather_start(j0 + 3, rows_b, sem_b)
            return carry

        last = chunks_per_worker - 2
        lax.fori_loop(0, chunks_per_worker // 2 - 1, body, 0)
        gather_wait(last, rows_a, sem_a)
        writeback(last, rows_a)
        gather_wait(last + 1, rows_b, sem_b)
        writeback(last + 1, rows_b)

    return lookup


def kernel(x, table):
    batch, hist = x.shape
    vocab, embed = table.shape
    hist_pad = -(-hist // 8) * 8
    total_pad = batch * hist_pad
    assert total_pad % (NUM_WORKERS * CHUNK) == 0
    chunks_per_worker = total_pad // (NUM_WORKERS * CHUNK)

    npad = hist_pad - hist
    pad_idx = (jnp.arange(batch * npad, dtype=jnp.int32) * 16381) % vocab
    xp = jnp.concatenate(
        [x.astype(jnp.int32), pad_idx.reshape(batch, npad)], axis=1)
    xp = xp.reshape(total_pad // CHUNK, CHUNK)
    lookup = _make_lookup(chunks_per_worker, embed)
    out = lookup(xp, table)
    return out.reshape(batch, hist_pad, embed)[:, :hist, :]


# speedup vs baseline: 7.4610x; 1.1303x over previous
import functools

import jax
import jax.numpy as jnp
from jax import lax
from jax.experimental import pallas as pl
from jax.experimental.pallas import tpu as pltpu
from jax.experimental.pallas import tpu_sc as plsc

NUM_WORKERS = 32
BGROUP = 2


def _make_lookup(batch, hist, hist_pad, embed):
    b_per_w = batch // NUM_WORKERS
    chunks_per_worker = b_per_w // BGROUP
    idx_per_w = b_per_w * hist_pad
    span = BGROUP * hist_pad

    mesh = plsc.VectorSubcoreMesh(core_axis_name="c", subcore_axis_name="s")

    @functools.partial(
        pl.kernel,
        out_type=jax.ShapeDtypeStruct((batch, hist, embed), jnp.float32),
        mesh=mesh,
        scratch_types=[
            pltpu.VMEM((idx_per_w,), jnp.int32),
            pltpu.VMEM((span, embed), jnp.float32),
            pltpu.VMEM((span, embed), jnp.float32),
            pltpu.SemaphoreType.DMA,
            pltpu.SemaphoreType.DMA,
        ],
        compiler_params=pltpu.CompilerParams(use_tc_tiling_on_sc=True),
    )
    def lookup(x_hbm, table_hbm, out_hbm, idx_v, rows_a, rows_b, sem_a, sem_b):
        num_cores = 2
        wid = lax.axis_index("s") * num_cores + lax.axis_index("c")
        b_base = wid * b_per_w

        pltpu.sync_copy(x_hbm.at[pl.ds(wid * idx_per_w, idx_per_w)], idx_v)

        def gather_start(j, buf, sem):
            pltpu.async_copy(
                table_hbm.at[idx_v.at[pl.ds(j * span, span)]], buf, sem)

        def gather_wait(j, buf, sem):
            pltpu.make_async_copy(
                table_hbm.at[idx_v.at[pl.ds(j * span, span)]], buf, sem).wait()

        def writeback(j, buf):
            b0 = b_base + j * BGROUP
            for g in range(BGROUP):
                pltpu.sync_copy(
                    buf.at[pl.ds(g * hist_pad, hist)], out_hbm.at[b0 + g])

        gather_start(0, rows_a, sem_a)
        gather_start(1, rows_b, sem_b)

        def body(jj, carry):
            j0 = 2 * jj
            gather_wait(j0, rows_a, sem_a)
            writeback(j0, rows_a)
            gather_start(j0 + 2, rows_a, sem_a)
            gather_wait(j0 + 1, rows_b, sem_b)
            writeback(j0 + 1, rows_b)
            gather_start(j0 + 3, rows_b, sem_b)
            return carry

        last = chunks_per_worker - 2
        lax.fori_loop(0, chunks_per_worker // 2 - 1, body, 0)
        gather_wait(last, rows_a, sem_a)
        writeback(last, rows_a)
        gather_wait(last + 1, rows_b, sem_b)
        writeback(last + 1, rows_b)

    return lookup


def kernel(x, table):
    batch, hist = x.shape
    vocab, embed = table.shape
    hist_pad = -(-hist // 8) * 8
    assert batch % NUM_WORKERS == 0 and (batch // NUM_WORKERS) % BGROUP == 0
    assert (BGROUP * hist_pad) % 8 == 0 and BGROUP * hist_pad <= 128

    npad = hist_pad - hist
    pad_idx = (jnp.arange(batch * npad, dtype=jnp.int32) * 16381) % vocab
    xp = jnp.concatenate(
        [x.astype(jnp.int32), pad_idx.reshape(batch, npad)], axis=1)
    xp = xp.reshape(batch * hist_pad)

    lookup = _make_lookup(batch, hist, hist_pad, embed)
    return lookup(xp, table)
